# baseline (speedup 1.0000x reference)
import functools

import numpy as np
import jax
import jax.numpy as jnp
from jax import lax
from jax.experimental import pallas as pl
from jax.experimental.pallas import tpu as pltpu
from jax.experimental.pallas import tpu_sc as plsc

NUM_FIELDS = 26
FIELD_SIZE = 38461
EMBED_DIM = 32
BATCH = 4096
NUM_IX = NUM_FIELDS * (NUM_FIELDS - 1) // 2
EMB_FLAT = NUM_FIELDS * EMBED_DIM

_CHUNK = 128


def _sc_gather(table_pad, idx2):
    info = plsc.get_sparse_core_info()
    nc, ns = info.num_cores, info.num_subcores
    n_chunks = idx2.shape[1] // _CHUNK
    rows_per_w = idx2.shape[1]
    mesh = plsc.VectorSubcoreMesh(core_axis_name="c", subcore_axis_name="s")

    @functools.partial(
        pl.kernel,
        mesh=mesh,
        compiler_params=pltpu.CompilerParams(use_tc_tiling_on_sc=True),
        out_type=jax.ShapeDtypeStruct((NUM_FIELDS, BATCH, 128), jnp.float32),
        scratch_types=[
            pltpu.VMEM((rows_per_w,), jnp.int32),
            pltpu.VMEM((2, _CHUNK, 128), jnp.float32),
            pltpu.SemaphoreType.DMA,
            pltpu.SemaphoreType.DMA,
        ],
    )
    def gather_kernel(tab_hbm, idx_hbm, out_hbm, idx_v, buf_v, gsem, wsem):
        wid = lax.axis_index("s") * nc + lax.axis_index("c")
        base = wid * rows_per_w
        pltpu.sync_copy(idx_hbm.at[wid], idx_v)
        gathers = []
        writes = []
        for c in range(n_chunks):
            p = c % 2
            gathers.append(pltpu.make_async_copy(
                tab_hbm.at[idx_v.at[pl.ds(c * _CHUNK, _CHUNK)]],
                buf_v.at[p],
                gsem,
            ))
            r0 = base + c * _CHUNK
            f = r0 // BATCH
            b0 = r0 % BATCH
            writes.append(pltpu.make_async_copy(
                buf_v.at[p],
                out_hbm.at[f, pl.ds(b0, _CHUNK)],
                wsem,
            ))
        gathers[0].start()
        for c in range(n_chunks):
            gathers[c].wait()
            writes[c].start()
            if c + 1 < n_chunks:
                if c >= 1:
                    writes[c - 1].wait()
                gathers[c + 1].start()
        writes[n_chunks - 2].wait()
        writes[n_chunks - 1].wait()

    return gather_kernel(table_pad, idx2)


def _mlp_body(e_ref, w1a_ref, w1b_ref, b1_ref, w2_ref, b2_ref, w3_ref, b3_ref,
              o_ref):
    v = e_ref[...]
    tb = v.shape[1]
    et = jnp.concatenate(
        [jnp.transpose(v[f])[:EMBED_DIM, :] for f in range(NUM_FIELDS)],
        axis=0)
    h = jnp.dot(w1a_ref[...], et, preferred_element_type=jnp.float32)
    off = 0
    for dlt in range(1, NUM_FIELDS):
        k = NUM_FIELDS - dlt
        rows = k * EMBED_DIM
        a = et[:rows, :] * et[dlt * EMBED_DIM:, :]
        p = jnp.sum(a.reshape(k, EMBED_DIM, tb), axis=1)
        h = h + jnp.dot(w1b_ref[:, off:off + k], p,
                        preferred_element_type=jnp.float32)
        off += k
    h = jnp.maximum(h + b1_ref[...], 0.0)
    h = jnp.dot(w2_ref[...], h, preferred_element_type=jnp.float32)
    h = jnp.maximum(h + b2_ref[...], 0.0)
    o = jnp.sum(h * w3_ref[...], axis=0) + b3_ref[0, 0]
    o_ref[...] = jax.nn.sigmoid(o)[None, None, :]


def _mlp_call(emb3, w1a, w1b, b1f, w2f, b2f, w3c, b3s, tb=512):
    grid = (BATCH // tb,)
    const = lambda i: (0, 0)
    out = pl.pallas_call(
        _mlp_body,
        grid=grid,
        in_specs=[
            pl.BlockSpec((NUM_FIELDS, tb, 128), lambda i: (0, i, 0)),
            pl.BlockSpec((256, EMB_FLAT), const),
            pl.BlockSpec((256, NUM_IX), const),
            pl.BlockSpec((256, 1), const),
            pl.BlockSpec((128, 256), const),
            pl.BlockSpec((128, 1), const),
            pl.BlockSpec((128, 1), const),
            pl.BlockSpec((1, 1), const),
        ],
        out_specs=pl.BlockSpec((1, 1, tb), lambda i: (i, 0, 0)),
        out_shape=jax.ShapeDtypeStruct((BATCH // tb, 1, tb), jnp.float32),
    )(emb3, w1a, w1b, b1f, w2f, b2f, w3c, b3s)
    return out.reshape(BATCH)


def _delta_perm():
    row, col = np.triu_indices(NUM_FIELDS, k=1)
    lut = {(i, j): n for n, (i, j) in enumerate(zip(row, col))}
    perm = [lut[(f, f + dlt)]
            for dlt in range(1, NUM_FIELDS)
            for f in range(NUM_FIELDS - dlt)]
    return np.asarray(perm, dtype=np.int32)


_PERM = _delta_perm()


def kernel(x, table, W1, b1, g1, be1, W2, b2, g2, be2, W3, b3):
    offsets = (np.arange(NUM_FIELDS) * FIELD_SIZE).astype(np.int32)
    idx = x.astype(jnp.int32).T + offsets[:, None]
    idx2 = idx.reshape(32, NUM_FIELDS * _CHUNK)
    table_pad = jnp.pad(table, ((0, 0), (0, 96)))

    emb3 = _sc_gather(table_pad, idx2)

    c = np.float32(1.0 / np.sqrt(1.0 + 1e-5))
    w1f = (W1 * (g1 * c)[None, :]).T
    b1f = ((b1 * g1 * c) + be1)[:, None]
    w1a = w1f[:, :EMB_FLAT]
    w1b = w1f[:, EMB_FLAT:][:, _PERM]
    w2f = (W2 * (g2 * c)[None, :]).T
    b2f = ((b2 * g2 * c) + be2)[:, None]
    b3s = b3.reshape(1, 1)

    return _mlp_call(emb3, w1a, w1b, b1f, w2f, b2f, W3, b3s)


# speedup vs baseline: 1.0527x; 1.0527x over previous
import functools

import numpy as np
import jax
import jax.numpy as jnp
from jax import lax
from jax.experimental import pallas as pl
from jax.experimental.pallas import tpu as pltpu
from jax.experimental.pallas import tpu_sc as plsc

NUM_FIELDS = 26
FIELD_SIZE = 38461
EMBED_DIM = 32
BATCH = 4096
NUM_IX = NUM_FIELDS * (NUM_FIELDS - 1) // 2
EMB_FLAT = NUM_FIELDS * EMBED_DIM

_CHUNK = 128


def _sc_gather(table, idx3):
    info = plsc.get_sparse_core_info()
    nc, ns = info.num_cores, info.num_subcores
    n_chunks = idx3.shape[1]
    rows_per_w = n_chunks * _CHUNK
    mesh = plsc.VectorSubcoreMesh(core_axis_name="c", subcore_axis_name="s")

    @functools.partial(
        pl.kernel,
        mesh=mesh,
        compiler_params=pltpu.CompilerParams(use_tc_tiling_on_sc=False),
        out_type=jax.ShapeDtypeStruct((NUM_FIELDS, BATCH, 128), jnp.float32),
        scratch_types=[
            pltpu.VMEM((n_chunks, _CHUNK), jnp.int32),
            pltpu.VMEM((rows_per_w, EMBED_DIM), jnp.float32),
            pltpu.SemaphoreType.DMA,
            pltpu.SemaphoreType.DMA,
        ],
    )
    def gather_kernel(tab_hbm, idx_hbm, out_hbm, idx_v, rows_v, sem, sem2):
        wid = lax.axis_index("s") * nc + lax.axis_index("c")
        base = wid * rows_per_w
        pltpu.sync_copy(idx_hbm.at[wid], idx_v)
        gathers = [
            pltpu.make_async_copy(
                tab_hbm.at[idx_v.at[c]],
                rows_v.at[pl.ds(c * _CHUNK, _CHUNK)],
                sem,
            )
            for c in range(n_chunks)
        ]
        for g in gathers:
            g.start()
        for g in gathers:
            g.wait()
        writes = []
        for c in range(n_chunks):
            r0 = base + c * _CHUNK
            f = r0 // BATCH
            b0 = r0 % BATCH
            writes.append(pltpu.make_async_copy(
                rows_v.at[pl.ds(c * _CHUNK, _CHUNK)],
                out_hbm.at[f, pl.ds(b0, _CHUNK), pl.ds(0, EMBED_DIM)],
                sem2,
            ))
        for wcp in writes:
            wcp.start()
        for wcp in writes:
            wcp.wait()

    return gather_kernel(table, idx3)


def _mlp_body(e_ref, w1a_ref, w1b_ref, b1_ref, w2_ref, b2_ref, w3_ref, b3_ref,
              o_ref):
    v = e_ref[...]
    tb = v.shape[1]
    et = jnp.concatenate(
        [jnp.transpose(v[f])[:EMBED_DIM, :] for f in range(NUM_FIELDS)],
        axis=0)
    h = jnp.dot(w1a_ref[...], et, preferred_element_type=jnp.float32)
    off = 0
    for dlt in range(1, NUM_FIELDS):
        k = NUM_FIELDS - dlt
        rows = k * EMBED_DIM
        a = et[:rows, :] * et[dlt * EMBED_DIM:, :]
        p = jnp.sum(a.reshape(k, EMBED_DIM, tb), axis=1)
        h = h + jnp.dot(w1b_ref[:, off:off + k], p,
                        preferred_element_type=jnp.float32)
        off += k
    h = jnp.maximum(h + b1_ref[...], 0.0)
    h = jnp.dot(w2_ref[...], h, preferred_element_type=jnp.float32)
    h = jnp.maximum(h + b2_ref[...], 0.0)
    o = jnp.sum(h * w3_ref[...], axis=0) + b3_ref[0, 0]
    o_ref[...] = jax.nn.sigmoid(o)[None, None, :]


def _mlp_call(emb3, w1a, w1b, b1f, w2f, b2f, w3c, b3s, tb=512):
    grid = (BATCH // tb,)
    const = lambda i: (0, 0)
    out = pl.pallas_call(
        _mlp_body,
        grid=grid,
        in_specs=[
            pl.BlockSpec((NUM_FIELDS, tb, 128), lambda i: (0, i, 0)),
            pl.BlockSpec((256, EMB_FLAT), const),
            pl.BlockSpec((256, NUM_IX), const),
            pl.BlockSpec((256, 1), const),
            pl.BlockSpec((128, 256), const),
            pl.BlockSpec((128, 1), const),
            pl.BlockSpec((128, 1), const),
            pl.BlockSpec((1, 1), const),
        ],
        out_specs=pl.BlockSpec((1, 1, tb), lambda i: (i, 0, 0)),
        out_shape=jax.ShapeDtypeStruct((BATCH // tb, 1, tb), jnp.float32),
    )(emb3, w1a, w1b, b1f, w2f, b2f, w3c, b3s)
    return out.reshape(BATCH)


def _delta_perm():
    row, col = np.triu_indices(NUM_FIELDS, k=1)
    lut = {(i, j): n for n, (i, j) in enumerate(zip(row, col))}
    perm = [lut[(f, f + dlt)]
            for dlt in range(1, NUM_FIELDS)
            for f in range(NUM_FIELDS - dlt)]
    return np.asarray(perm, dtype=np.int32)


_PERM = _delta_perm()


def kernel(x, table, W1, b1, g1, be1, W2, b2, g2, be2, W3, b3):
    offsets = (np.arange(NUM_FIELDS) * FIELD_SIZE).astype(np.int32)
    idx = x.astype(jnp.int32).T + offsets[:, None]
    idx3 = idx.reshape(32, NUM_FIELDS, _CHUNK)

    emb3 = _sc_gather(table, idx3)

    c = np.float32(1.0 / np.sqrt(1.0 + 1e-5))
    w1f = (W1 * (g1 * c)[None, :]).T
    b1f = ((b1 * g1 * c) + be1)[:, None]
    w1a = w1f[:, :EMB_FLAT]
    w1b = w1f[:, EMB_FLAT:][:, _PERM]
    w2f = (W2 * (g2 * c)[None, :]).T
    b2f = ((b2 * g2 * c) + be2)[:, None]
    b3s = b3.reshape(1, 1)

    return _mlp_call(emb3, w1a, w1b, b1f, w2f, b2f, W3, b3s)


# speedup vs baseline: 1.0555x; 1.0027x over previous
import functools

import numpy as np
import jax
import jax.numpy as jnp
from jax import lax
from jax.experimental import pallas as pl
from jax.experimental.pallas import tpu as pltpu
from jax.experimental.pallas import tpu_sc as plsc

NUM_FIELDS = 26
FIELD_SIZE = 38461
EMBED_DIM = 32
BATCH = 4096
NUM_IX = NUM_FIELDS * (NUM_FIELDS - 1) // 2
EMB_FLAT = NUM_FIELDS * EMBED_DIM

_CHUNK = 128


def _sc_gather(table, idx3):
    info = plsc.get_sparse_core_info()
    nc, ns = info.num_cores, info.num_subcores
    n_chunks = idx3.shape[1]
    rows_per_w = n_chunks * _CHUNK
    mesh = plsc.VectorSubcoreMesh(core_axis_name="c", subcore_axis_name="s")

    @functools.partial(
        pl.kernel,
        mesh=mesh,
        compiler_params=pltpu.CompilerParams(use_tc_tiling_on_sc=False),
        out_type=jax.ShapeDtypeStruct((NUM_FIELDS, BATCH, 128), jnp.float32),
        scratch_types=[
            pltpu.VMEM((n_chunks, _CHUNK), jnp.int32),
            pltpu.VMEM((rows_per_w, EMBED_DIM), jnp.float32),
            pltpu.SemaphoreType.DMA,
            pltpu.SemaphoreType.DMA,
        ],
    )
    def gather_kernel(tab_hbm, idx_hbm, out_hbm, idx_v, rows_v, sem, sem2):
        wid = lax.axis_index("s") * nc + lax.axis_index("c")
        base = wid * rows_per_w
        pltpu.sync_copy(idx_hbm.at[wid], idx_v)
        gathers = [
            pltpu.make_async_copy(
                tab_hbm.at[idx_v.at[c]],
                rows_v.at[pl.ds(c * _CHUNK, _CHUNK)],
                sem,
            )
            for c in range(n_chunks)
        ]
        for g in gathers:
            g.start()
        for g in gathers:
            g.wait()
        writes = []
        for c in range(n_chunks):
            r0 = base + c * _CHUNK
            f = r0 // BATCH
            b0 = r0 % BATCH
            writes.append(pltpu.make_async_copy(
                rows_v.at[pl.ds(c * _CHUNK, _CHUNK)],
                out_hbm.at[f, pl.ds(b0, _CHUNK), pl.ds(0, EMBED_DIM)],
                sem2,
            ))
        for wcp in writes:
            wcp.start()
        for wcp in writes:
            wcp.wait()

    return gather_kernel(table, idx3)


def _mlp_body(e_ref, w1a_ref, w1b_ref, b1_ref, w2_ref, b2_ref, w3_ref, b3_ref,
              o_ref):
    v = e_ref[...]
    tb = v.shape[1]
    et = jnp.concatenate(
        [jnp.transpose(v[f])[:EMBED_DIM, :] for f in range(NUM_FIELDS)],
        axis=0)
    h = jnp.dot(w1a_ref[...], et, preferred_element_type=jnp.float32)
    off = 0
    for dlt in range(1, NUM_FIELDS):
        k = NUM_FIELDS - dlt
        rows = k * EMBED_DIM
        a = et[:rows, :] * et[dlt * EMBED_DIM:, :]
        p = jnp.sum(a.reshape(k, EMBED_DIM, tb), axis=1)
        h = h + jnp.dot(w1b_ref[:, off:off + k], p,
                        preferred_element_type=jnp.float32)
        off += k
    h = jnp.maximum(h + b1_ref[...], 0.0)
    h = jnp.dot(w2_ref[...], h, preferred_element_type=jnp.float32)
    h = jnp.maximum(h + b2_ref[...], 0.0)
    o = jnp.sum(h * w3_ref[...], axis=0) + b3_ref[0, 0]
    o_ref[...] = jax.nn.sigmoid(o)[None, None, :]


def _mlp_call(emb3, w1a, w1b, b1f, w2f, b2f, w3c, b3s, tb=1024):
    grid = (BATCH // tb,)
    const = lambda i: (0, 0)
    out = pl.pallas_call(
        _mlp_body,
        grid=grid,
        in_specs=[
            pl.BlockSpec((NUM_FIELDS, tb, 128), lambda i: (0, i, 0)),
            pl.BlockSpec((256, EMB_FLAT), const),
            pl.BlockSpec((256, NUM_IX), const),
            pl.BlockSpec((256, 1), const),
            pl.BlockSpec((128, 256), const),
            pl.BlockSpec((128, 1), const),
            pl.BlockSpec((128, 1), const),
            pl.BlockSpec((1, 1), const),
        ],
        out_specs=pl.BlockSpec((1, 1, tb), lambda i: (i, 0, 0)),
        out_shape=jax.ShapeDtypeStruct((BATCH // tb, 1, tb), jnp.float32),
    )(emb3, w1a, w1b, b1f, w2f, b2f, w3c, b3s)
    return out.reshape(BATCH)


def _delta_perm():
    row, col = np.triu_indices(NUM_FIELDS, k=1)
    lut = {(i, j): n for n, (i, j) in enumerate(zip(row, col))}
    perm = [lut[(f, f + dlt)]
            for dlt in range(1, NUM_FIELDS)
            for f in range(NUM_FIELDS - dlt)]
    return np.asarray(perm, dtype=np.int32)


_PERM = _delta_perm()


def kernel(x, table, W1, b1, g1, be1, W2, b2, g2, be2, W3, b3):
    offsets = (np.arange(NUM_FIELDS) * FIELD_SIZE).astype(np.int32)
    idx = x.astype(jnp.int32).T + offsets[:, None]
    idx3 = idx.reshape(32, NUM_FIELDS, _CHUNK)

    emb3 = _sc_gather(table, idx3)

    c = np.float32(1.0 / np.sqrt(1.0 + 1e-5))
    w1f = (W1 * (g1 * c)[None, :]).T
    b1f = ((b1 * g1 * c) + be1)[:, None]
    w1a = w1f[:, :EMB_FLAT]
    w1b = w1f[:, EMB_FLAT:][:, _PERM]
    w2f = (W2 * (g2 * c)[None, :]).T
    b2f = ((b2 * g2 * c) + be2)[:, None]
    b3s = b3.reshape(1, 1)

    return _mlp_call(emb3, w1a, w1b, b1f, w2f, b2f, W3, b3s)
